# baseline (speedup 1.0000x reference)
import functools

import jax
import jax.numpy as jnp
from jax import lax
from jax.experimental import pallas as pl
from jax.experimental.pallas import tpu as pltpu
from jax.experimental.pallas import tpu_sc as plsc

TOKENS = 32768
HIDDEN = 768
MAX_POS = 8192
TYPE_VOCAB = 2
LANES = 16
NCORES = 2
NSUB = 16
NWORK = NCORES * NSUB
TPW = TOKENS // NWORK
CHUNK = 32
NCHUNK = TPW // CHUNK
NJ = HIDDEN // LANES

_mesh = plsc.VectorSubcoreMesh(core_axis_name="c", subcore_axis_name="s")


@functools.partial(
    pl.kernel,
    mesh=_mesh,
    out_type=jax.ShapeDtypeStruct((TOKENS, HIDDEN), jnp.float32),
    scratch_types=[
        pltpu.VMEM((TPW,), jnp.int32),
        pltpu.VMEM((TPW,), jnp.int32),
        pltpu.VMEM((TPW,), jnp.int32),
        pltpu.VMEM((TYPE_VOCAB, HIDDEN), jnp.float32),
        pltpu.VMEM((CHUNK, HIDDEN), jnp.float32),
        pltpu.VMEM((CHUNK, HIDDEN), jnp.float32),
        pltpu.VMEM((CHUNK, HIDDEN), jnp.float32),
        pltpu.VMEM((CHUNK, HIDDEN), jnp.float32),
        pltpu.VMEM((LANES,), jnp.float32),
        pltpu.SemaphoreType.DMA,
        pltpu.SemaphoreType.DMA,
        pltpu.SemaphoreType.DMA,
        pltpu.SemaphoreType.DMA,
    ],
)
def _emb5_kernel(ids_w_hbm, ids_p_hbm, pos_tab_hbm, ids_t_hbm, type_tab_hbm,
                 scal_hbm, weight_hbm, out_hbm,
                 idx_w, idx_p, idx_t, ttab, wbuf0, wbuf1, pbuf0, pbuf1, svmem,
                 sem0, sem1, osem0, osem1):
    wbuf = (wbuf0, wbuf1)
    pbuf = (pbuf0, pbuf1)
    sem = (sem0, sem1)
    osem = (osem0, osem1)

    wid = lax.axis_index("s") * NCORES + lax.axis_index("c")
    base = wid * TPW
    pltpu.sync_copy(scal_hbm, svmem)
    svec = svmem[...]
    pltpu.sync_copy(type_tab_hbm, ttab)
    bslice = pl.ds(pl.multiple_of(base, TPW), TPW)
    pltpu.sync_copy(ids_w_hbm.at[bslice], idx_w)
    pltpu.sync_copy(ids_p_hbm.at[bslice], idx_p)
    pltpu.sync_copy(ids_t_hbm.at[bslice], idx_t)

    def fire(c, b):
        loc = pl.multiple_of(c * CHUNK, CHUNK)
        pltpu.async_copy(weight_hbm.at[idx_w.at[pl.ds(loc, CHUNK)]],
                         wbuf[b], sem[b])
        pltpu.async_copy(pos_tab_hbm.at[idx_p.at[pl.ds(loc, CHUNK)]],
                         pbuf[b], sem[b])

    def wait_gathers(c, b):
        loc = pl.multiple_of(c * CHUNK, CHUNK)
        pltpu.make_async_copy(weight_hbm.at[idx_w.at[pl.ds(loc, CHUNK)]],
                              wbuf[b], sem[b]).wait()
        pltpu.make_async_copy(pos_tab_hbm.at[idx_p.at[pl.ds(loc, CHUNK)]],
                              pbuf[b], sem[b]).wait()

    def out_slice(c):
        off = pl.multiple_of(base + c * CHUNK, CHUNK)
        return out_hbm.at[pl.ds(off, CHUNK)]

    fire(0, 0)

    def pair_body(g2, carry):
        for b in range(2):
            c = g2 * 2 + b
            wait_gathers(c, b)

            @pl.when(jnp.logical_and(c >= 1, c + 1 < NCHUNK))
            def _():
                pltpu.make_async_copy(wbuf[1 - b], out_slice(c - 1),
                                      osem[1 - b]).wait()

            @pl.when(c + 1 < NCHUNK)
            def _():
                fire(c + 1, 1 - b)

            loc = pl.multiple_of(c * CHUNK, CHUNK)
            fvec = tuple(
                idx_t[pl.ds(loc + g * LANES, LANES)].astype(jnp.float32)
                for g in range(CHUNK // LANES))

            def jbody(j, cj):
                sl = pl.ds(j * LANES, LANES)
                t0 = ttab[0, sl]
                d = ttab[1, sl] - t0

                def tok(t, c2):
                    tvec = jnp.full((LANES, 1), t, dtype=jnp.int32)
                    for g in range(CHUNK // LANES):
                        fb = lax.gather(
                            fvec[g], tvec,
                            dimension_numbers=lax.GatherDimensionNumbers(
                                offset_dims=(), collapsed_slice_dims=(0,),
                                start_index_map=(0,)),
                            slice_sizes=(1,),
                            mode=lax.GatherScatterMode.PROMISE_IN_BOUNDS)
                        row = t + g * LANES
                        wbuf[b][row, sl] = (wbuf[b][row, sl] * svec
                                            + pbuf[b][row, sl] + t0 + fb * d)
                    return c2

                lax.fori_loop(0, LANES, tok, 0, unroll=4)
                return cj

            lax.fori_loop(0, NJ, jbody, 0)
            pltpu.async_copy(wbuf[b], out_slice(c), osem[b])
        return carry

    lax.fori_loop(0, NCHUNK // 2, pair_body, 0)
    pltpu.make_async_copy(wbuf[0], out_slice(NCHUNK - 2), osem[0]).wait()
    pltpu.make_async_copy(wbuf[1], out_slice(NCHUNK - 1), osem[1]).wait()


def kernel(input, combo_position_ids, position_encoding, combo_tokens_type_ids,
           token_type_embedding, input_embedding_scalar, weight):
    ids_w = input.astype(jnp.int32)
    ids_p = combo_position_ids.astype(jnp.int32)
    ids_t = combo_tokens_type_ids.astype(jnp.int32)
    sv = jnp.full((LANES,), input_embedding_scalar, dtype=jnp.float32)
    return _emb5_kernel(ids_w, ids_p, position_encoding, ids_t,
                        token_type_embedding, sv, weight)


# speedup vs baseline: 2.6800x; 2.6800x over previous
import functools

import jax
import jax.numpy as jnp
from jax import lax
from jax.experimental import pallas as pl
from jax.experimental.pallas import tpu as pltpu
from jax.experimental.pallas import tpu_sc as plsc

TOKENS = 32768
HIDDEN = 768
MAX_POS = 8192
TYPE_VOCAB = 2
LANES = 16
NCORES = 2
NSUB = 16
NWORK = NCORES * NSUB
TPW = TOKENS // NWORK
CHUNK = 32
NCHUNK = TPW // CHUNK
NJ = HIDDEN // LANES
BP = 1024

_mesh = plsc.VectorSubcoreMesh(core_axis_name="c", subcore_axis_name="s")


def _pt_body(pos_ref, typ_ref, out_ref):
    p = pos_ref[...]
    t = typ_ref[...]
    out_ref[...] = (p[:, None, :] + t[None, :, :]).reshape(
        BP * TYPE_VOCAB, HIDDEN)


_pt_build = pl.pallas_call(
    _pt_body,
    grid=(MAX_POS // BP,),
    in_specs=[pl.BlockSpec((BP, HIDDEN), lambda i: (i, 0)),
              pl.BlockSpec((TYPE_VOCAB, HIDDEN), lambda i: (0, 0))],
    out_specs=pl.BlockSpec((BP * TYPE_VOCAB, HIDDEN), lambda i: (i, 0)),
    out_shape=jax.ShapeDtypeStruct((MAX_POS * TYPE_VOCAB, HIDDEN),
                                   jnp.float32),
)


@functools.partial(
    pl.kernel,
    mesh=_mesh,
    out_type=jax.ShapeDtypeStruct((TOKENS, HIDDEN), jnp.float32),
    scratch_types=[
        pltpu.VMEM((TPW,), jnp.int32),
        pltpu.VMEM((TPW,), jnp.int32),
        pltpu.VMEM((CHUNK, HIDDEN), jnp.float32),
        pltpu.VMEM((CHUNK, HIDDEN), jnp.float32),
        pltpu.VMEM((CHUNK, HIDDEN), jnp.float32),
        pltpu.VMEM((CHUNK, HIDDEN), jnp.float32),
        pltpu.VMEM((LANES,), jnp.float32),
        pltpu.SemaphoreType.DMA,
        pltpu.SemaphoreType.DMA,
        pltpu.SemaphoreType.DMA,
        pltpu.SemaphoreType.DMA,
    ],
)
def _emb2_kernel(ids_w_hbm, ids_pt_hbm, pt_tab_hbm, scal_hbm, weight_hbm,
                 out_hbm,
                 idx_w, idx_pt, wbuf0, wbuf1, ptbuf0, ptbuf1, svmem,
                 sem0, sem1, osem0, osem1):
    wbuf = (wbuf0, wbuf1)
    ptbuf = (ptbuf0, ptbuf1)
    sem = (sem0, sem1)
    osem = (osem0, osem1)

    wid = lax.axis_index("s") * NCORES + lax.axis_index("c")
    base = wid * TPW
    pltpu.sync_copy(scal_hbm, svmem)
    svec = svmem[...]
    pltpu.sync_copy(ids_w_hbm.at[pl.ds(pl.multiple_of(base, TPW), TPW)], idx_w)
    pltpu.sync_copy(ids_pt_hbm.at[pl.ds(pl.multiple_of(base, TPW), TPW)], idx_pt)

    def fire(c, b):
        loc = pl.multiple_of(c * CHUNK, CHUNK)
        pltpu.async_copy(weight_hbm.at[idx_w.at[pl.ds(loc, CHUNK)]],
                         wbuf[b], sem[b])
        pltpu.async_copy(pt_tab_hbm.at[idx_pt.at[pl.ds(loc, CHUNK)]],
                         ptbuf[b], sem[b])

    def wait_gathers(c, b):
        loc = pl.multiple_of(c * CHUNK, CHUNK)
        pltpu.make_async_copy(weight_hbm.at[idx_w.at[pl.ds(loc, CHUNK)]],
                              wbuf[b], sem[b]).wait()
        pltpu.make_async_copy(pt_tab_hbm.at[idx_pt.at[pl.ds(loc, CHUNK)]],
                              ptbuf[b], sem[b]).wait()

    def out_slice(c):
        off = pl.multiple_of(base + c * CHUNK, CHUNK)
        return out_hbm.at[pl.ds(off, CHUNK)]

    fire(0, 0)

    def pair_body(g2, carry):
        for b in range(2):
            c = g2 * 2 + b
            wait_gathers(c, b)

            @pl.when(jnp.logical_and(c >= 1, c + 1 < NCHUNK))
            def _():
                pltpu.make_async_copy(wbuf[1 - b], out_slice(c - 1),
                                      osem[1 - b]).wait()

            @pl.when(c + 1 < NCHUNK)
            def _():
                fire(c + 1, 1 - b)

            def tok(t, c2):
                for j in range(NJ):
                    sl = pl.ds(j * LANES, LANES)
                    wbuf[b][t, sl] = wbuf[b][t, sl] * svec + ptbuf[b][t, sl]
                return c2

            lax.fori_loop(0, CHUNK, tok, 0)
            pltpu.async_copy(wbuf[b], out_slice(c), osem[b])
        return carry

    lax.fori_loop(0, NCHUNK // 2, pair_body, 0)
    pltpu.make_async_copy(wbuf[0], out_slice(NCHUNK - 2), osem[0]).wait()
    pltpu.make_async_copy(wbuf[1], out_slice(NCHUNK - 1), osem[1]).wait()


def kernel(input, combo_position_ids, position_encoding, combo_tokens_type_ids,
           token_type_embedding, input_embedding_scalar, weight):
    ids_w = input.astype(jnp.int32)
    ids_pt = (combo_position_ids.astype(jnp.int32) * TYPE_VOCAB
              + combo_tokens_type_ids.astype(jnp.int32))
    pt_tab = _pt_build(position_encoding, token_type_embedding)
    sv = jnp.full((LANES,), input_embedding_scalar, dtype=jnp.float32)
    return _emb2_kernel(ids_w, ids_pt, pt_tab, sv, weight)
